# baseline (speedup 1.0000x reference)
import functools

import jax
import jax.numpy as jnp
from jax import lax
from jax.experimental import pallas as pl
from jax.experimental.pallas import tpu as pltpu
from jax.experimental.pallas import tpu_sc as plsc

ALPHA_C = 1.5
S = 2048
H = 12
QTOT = H * S
BATCH = 2
NW = 32
Q_SC = 7168
Q_TC = QTOT - Q_SC
QBLK = 1024
NQ = Q_TC // QBLK
RPW = Q_SC // NW
CH = 16
NCH = RPW // CH


def _tc_body(x_ref, o_ref, acc_ref):
    i = pl.program_id(1)

    @pl.when(i == 0)
    def _init():
        acc_ref[...] = jnp.zeros_like(acc_ref)

    x = x_ref[0]
    part = jnp.zeros((8, S), jnp.float32)
    for k in range(QBLK // 8):
        part = part + x[k * 8:(k + 1) * 8, :]
    acc_ref[...] += part

    @pl.when(i == NQ - 1)
    def _final():
        o_ref[0] = jnp.sum(acc_ref[...], axis=0, keepdims=True)


def _sc_partial_fn(x_hbm, out_hbm, buf_v, acc_v, sem0, sem1):
    wid = lax.axis_index("s") * 2 + lax.axis_index("c")
    row0 = Q_TC + wid * RPW
    sems = (sem0, sem1)
    zero = jnp.zeros((16,), jnp.float32)
    for b in range(BATCH):
        for j in range(S // 16):
            acc_v[pl.ds(j * 16, 16)] = zero
        cps = [None, None]
        cps[0] = pltpu.async_copy(
            x_hbm.at[b, pl.ds(row0, CH), :], buf_v.at[0], sems[0])
        for c in range(NCH):
            cur = c % 2
            nxt = (c + 1) % 2
            if c + 1 < NCH:
                cps[nxt] = pltpu.async_copy(
                    x_hbm.at[b, pl.ds(row0 + (c + 1) * CH, CH), :],
                    buf_v.at[nxt], sems[nxt])
            cps[cur].wait()
            bufc = buf_v.at[cur]

            @plsc.parallel_loop(0, S // 16, step=1)
            def _col(j):
                sl = pl.ds(j * 16, 16)
                v = bufc[0, sl]
                for r in range(1, CH):
                    v = v + bufc[r, sl]
                plsc.addupdate(acc_v.at[sl], v)
        pltpu.sync_copy(acc_v, out_hbm.at[b, wid])


def _combine_body(tcp_ref, scp_ref, m_ref, o_ref):
    colsum = tcp_ref[0] + jnp.sum(scp_ref[0], axis=0, keepdims=True)
    scores = colsum * jnp.float32(1.0 / QTOT)
    m = m_ref[0]
    sep_i = (jnp.sum(m) - 1.0).astype(jnp.int32)
    idx = jax.lax.broadcasted_iota(jnp.int32, (1, S), 1)
    valid = (idx >= 1) & (idx <= sep_i - 1)
    n = (sep_i - 1).astype(jnp.float32)
    mean = jnp.sum(jnp.where(valid, scores, 0.0)) / n
    dev = jnp.where(valid, scores - mean, 0.0)
    var = jnp.sum(dev * dev) / (n - 1.0)
    std = jnp.sqrt(var)
    lo = mean - jnp.float32(ALPHA_C) * std
    hi = mean + jnp.float32(ALPHA_C) * std
    keep = valid & (scores >= lo) & (scores <= hi)
    out = jnp.where(keep | (idx == 0) | (idx == sep_i), 1.0, 0.0)
    o_ref[0] = out.astype(jnp.float32)


_sc_partial = functools.partial(
    pl.kernel,
    out_type=jax.ShapeDtypeStruct((BATCH, NW, S), jnp.float32),
    mesh=plsc.VectorSubcoreMesh(core_axis_name="c", subcore_axis_name="s"),
    scratch_types=[
        pltpu.VMEM((2, CH, S), jnp.float32),
        pltpu.VMEM((S,), jnp.float32),
        pltpu.SemaphoreType.DMA,
        pltpu.SemaphoreType.DMA,
    ],
)(_sc_partial_fn)


@jax.jit
def kernel(layer_attention_probes, mask):
    b = layer_attention_probes.shape[0]
    x3 = layer_attention_probes.reshape(b, QTOT, S)
    mask3 = mask.reshape(b, 1, S)

    tc_partial = pl.pallas_call(
        _tc_body,
        grid=(b, NQ),
        in_specs=[pl.BlockSpec((1, QBLK, S), lambda bi, qi: (bi, qi, 0))],
        out_specs=pl.BlockSpec((1, 1, S), lambda bi, qi: (bi, 0, 0)),
        out_shape=jax.ShapeDtypeStruct((b, 1, S), jnp.float32),
        scratch_shapes=[pltpu.VMEM((8, S), jnp.float32)],
        compiler_params=pltpu.CompilerParams(
            dimension_semantics=("arbitrary", "arbitrary"),
        ),
    )(x3)

    sc_partial = _sc_partial(x3)

    out = pl.pallas_call(
        _combine_body,
        grid=(b,),
        in_specs=[
            pl.BlockSpec((1, 1, S), lambda bi: (bi, 0, 0)),
            pl.BlockSpec((1, NW, S), lambda bi: (bi, 0, 0)),
            pl.BlockSpec((1, 1, S), lambda bi: (bi, 0, 0)),
        ],
        out_specs=pl.BlockSpec((1, 1, S), lambda bi: (bi, 0, 0)),
        out_shape=jax.ShapeDtypeStruct((b, 1, S), jnp.float32),
    )(tc_partial, sc_partial, mask3)
    return out.reshape(b, S)


# speedup vs baseline: 1.1811x; 1.1811x over previous
import jax
import jax.numpy as jnp
from jax.experimental import pallas as pl
from jax.experimental.pallas import tpu as pltpu

ALPHA_C = 1.5
S = 2048
H = 12
QTOT = H * S
QBLK = 1024
NQ = QTOT // (2 * QBLK)


def _body(xa_ref, xb_ref, m_ref, o_ref, acc_ref):
    i = pl.program_id(1)

    @pl.when(i == 0)
    def _init():
        acc_ref[...] = jnp.zeros_like(acc_ref)

    part = jnp.zeros((8, S), jnp.float32)
    xa = xa_ref[0]
    xb = xb_ref[0]
    for k in range(QBLK // 8):
        part = part + xa[k * 8:(k + 1) * 8, :]
    for k in range(QBLK // 8):
        part = part + xb[k * 8:(k + 1) * 8, :]
    acc_ref[...] += part

    @pl.when(i == NQ - 1)
    def _epilogue():
        colsum = jnp.sum(acc_ref[...], axis=0, keepdims=True)
        scores = colsum * jnp.float32(1.0 / QTOT)
        m = m_ref[0]
        sep_i = (jnp.sum(m) - 1.0).astype(jnp.int32)
        idx = jax.lax.broadcasted_iota(jnp.int32, (1, S), 1)
        valid = (idx >= 1) & (idx <= sep_i - 1)
        n = (sep_i - 1).astype(jnp.float32)
        mean = jnp.sum(jnp.where(valid, scores, 0.0)) / n
        dev = jnp.where(valid, scores - mean, 0.0)
        var = jnp.sum(dev * dev) / (n - 1.0)
        std = jnp.sqrt(var)
        lo = mean - jnp.float32(ALPHA_C) * std
        hi = mean + jnp.float32(ALPHA_C) * std
        keep = valid & (scores >= lo) & (scores <= hi)
        out = jnp.where(keep | (idx == 0) | (idx == sep_i), 1.0, 0.0)
        o_ref[0] = out.astype(jnp.float32)


@jax.jit
def kernel(layer_attention_probes, mask):
    b = layer_attention_probes.shape[0]
    x3 = layer_attention_probes.reshape(b, QTOT, S)
    mask3 = mask.reshape(b, 1, S)
    out = pl.pallas_call(
        _body,
        grid=(b, NQ),
        in_specs=[
            pl.BlockSpec((1, QBLK, S), lambda bi, qi: (bi, qi, 0)),
            pl.BlockSpec((1, QBLK, S), lambda bi, qi: (bi, qi + NQ, 0)),
            pl.BlockSpec((1, 1, S), lambda bi, qi: (bi, 0, 0)),
        ],
        out_specs=pl.BlockSpec((1, 1, S), lambda bi, qi: (bi, 0, 0)),
        out_shape=jax.ShapeDtypeStruct((b, 1, S), jnp.float32),
        scratch_shapes=[pltpu.VMEM((8, S), jnp.float32)],
        compiler_params=pltpu.CompilerParams(
            dimension_semantics=("arbitrary", "arbitrary"),
        ),
    )(x3, x3, mask3)
    return out.reshape(b, S)
